# Initial kernel scaffold; baseline (speedup 1.0000x reference)
import jax
import jax.numpy as jnp
from jax.experimental import pallas as pl
from jax.experimental.pallas import tpu as pltpu

_F = 4
_L = 8
_DELTA_AGG = 0.5
_DELTA_DIS = 1.5
_W_AGG = 1.0
_W_DIS = 1.0


def _scalar_safe_sqrt(sq):
    safe = jnp.where(sq == 0.0, 1.0, sq)
    return jnp.where(sq == 0.0, 0.0, jnp.sqrt(safe))


def _loss_body(inst_ref, kern_ref, tm_ref, emb_ref, out_ref):
    inst = inst_ref[0]
    tm = tm_ref[0] > 0.5
    kn = kern_ref[0] > 0.5
    ik = jnp.where(tm & kn, inst, 0)
    ifl = jnp.where(tm, inst, 0)

    e = [emb_ref[0, f] for f in range(_F)]

    cnt_k0 = jnp.sum((ik == 0).astype(jnp.float32))
    cnt_k = [cnt_k0]
    cnt_f = [jnp.float32(0.0)]
    sums = [[jnp.float32(0.0)] * _F]
    for c in range(1, _L):
        mk = ik == c
        mf = ifl == c
        cnt_k.append(jnp.sum(mk.astype(jnp.float32)))
        cnt_f.append(jnp.sum(mf.astype(jnp.float32)))
        sums.append([jnp.sum(jnp.where(mk, e[f], 0.0)) for f in range(_F)])

    present = [(cnt_k[c] > 0.0).astype(jnp.float32) for c in range(_L)]
    num_inst = present[0]
    for c in range(1, _L):
        num_inst = num_inst + present[c]
    run = jnp.float32(0.0)
    valid = []
    for c in range(_L):
        run = run + present[c]
        rank = run - 1.0
        valid.append(present[c] * (rank >= 1.0).astype(jnp.float32))

    m = [[jnp.float32(0.0)] * _F]
    for c in range(1, _L):
        denom = jnp.maximum(cnt_k[c], 1.0)
        m.append([sums[c][f] / denom for f in range(_F)])
    msq = []
    for c in range(_L):
        s = m[c][0] * m[c][0]
        for f in range(1, _F):
            s = s + m[c][f] * m[c][f]
        msq.append(s)

    w = [jnp.float32(0.0)]
    for c in range(1, _L):
        w.append(valid[c] / jnp.maximum(cnt_f[c], 1.0))

    msel = [jnp.where(ifl == 1, m[1][f], 0.0) for f in range(_F)]
    msqsel = jnp.where(ifl == 1, msq[1], 0.0)
    wsel = jnp.where(ifl == 1, w[1], 0.0)
    for c in range(2, _L):
        mc = ifl == c
        msel = [jnp.where(mc, m[c][f], msel[f]) for f in range(_F)]
        msqsel = jnp.where(mc, msq[c], msqsel)
        wsel = jnp.where(mc, w[c], wsel)

    dot = e[0] * msel[0]
    esq = e[0] * e[0]
    for f in range(1, _F):
        dot = dot + e[f] * msel[f]
        esq = esq + e[f] * e[f]
    d2 = jnp.maximum(esq - 2.0 * dot + msqsel, 0.0)
    dist = jnp.where(d2 > 0.0, jnp.sqrt(jnp.where(d2 > 0.0, d2, 1.0)), 0.0)
    t = jnp.maximum(dist - _DELTA_AGG, 0.0)
    v = jnp.log(t * t + 1.0)
    agg_num = jnp.sum(v * wsel)
    l_agg = agg_num / jnp.maximum(num_inst - 1.0, 1.0)

    dis_num = jnp.float32(0.0)
    pm_sum = jnp.float32(0.0)
    for i in range(_L):
        for j in range(i + 1, _L):
            dsq = jnp.float32(0.0)
            for f in range(_F):
                dd = m[i][f] - m[j][f]
                dsq = dsq + dd * dd
            d = _scalar_safe_sqrt(dsq)
            vij = jnp.log(jnp.maximum(2.0 * _DELTA_DIS - d, 0.0) ** 2 + 1.0)
            pm_ij = valid[i] * valid[j]
            dis_num = dis_num + 2.0 * vij * pm_ij
            pm_sum = pm_sum + 2.0 * pm_ij
    l_dis = dis_num / jnp.maximum(pm_sum, 1.0)

    reg = jnp.float32(0.0)
    for c in range(_L):
        reg = reg + jnp.log(_scalar_safe_sqrt(msq[c]) + 1.0)
    l_reg = reg / jnp.maximum(num_inst, 1.0) * 0.001

    total = _W_AGG * l_agg + _W_DIS * l_dis + l_reg
    out_ref[0, 0] = jnp.where(num_inst <= 1.0, 0.0, total)


def kernel(emb, instance, kernel, training_mask):
    B, F, H, W = emb.shape
    instance = instance.astype(jnp.int32)
    out = pl.pallas_call(
        _loss_body,
        grid=(B,),
        in_specs=[
            pl.BlockSpec((1, H, W), lambda b: (b, 0, 0)),
            pl.BlockSpec((1, H, W), lambda b: (b, 0, 0)),
            pl.BlockSpec((1, H, W), lambda b: (b, 0, 0)),
            pl.BlockSpec((1, F, H, W), lambda b: (b, 0, 0, 0)),
        ],
        out_specs=pl.BlockSpec(
            (1, 1), lambda b: (b, 0), memory_space=pltpu.SMEM
        ),
        out_shape=jax.ShapeDtypeStruct((B, 1), jnp.float32),
        compiler_params=pltpu.CompilerParams(
            dimension_semantics=("arbitrary",),
        ),
    )(instance, kernel, training_mask, emb)
    return out.reshape(B)


# speedup vs baseline: 5.5458x; 5.5458x over previous
import jax
import jax.numpy as jnp
from jax.experimental import pallas as pl
from jax.experimental.pallas import tpu as pltpu

_F = 4
_L = 8
_DELTA_AGG = 0.5
_DELTA_DIS = 1.5
_W_AGG = 1.0
_W_DIS = 1.0


def _scalar_safe_sqrt(sq):
    safe = jnp.where(sq == 0.0, 1.0, sq)
    return jnp.where(sq == 0.0, 0.0, jnp.sqrt(safe))


def _loss_body(inst_ref, kern_ref, tm_ref, emb_ref, out_ref):
    inst = inst_ref[0]
    tm = tm_ref[0] > 0.5
    kn = kern_ref[0] > 0.5
    ik = jnp.where(tm & kn, inst, 0)
    ifl = jnp.where(tm, inst, 0)

    e = [emb_ref[0, f] for f in range(_F)]

    cnt_k0 = jnp.sum((ik == 0).astype(jnp.float32))
    cnt_k = [cnt_k0]
    cnt_f = [jnp.float32(0.0)]
    sums = [[jnp.float32(0.0)] * _F]
    for c in range(1, _L):
        mk = ik == c
        mf = ifl == c
        cnt_k.append(jnp.sum(mk.astype(jnp.float32)))
        cnt_f.append(jnp.sum(mf.astype(jnp.float32)))
        sums.append([jnp.sum(jnp.where(mk, e[f], 0.0)) for f in range(_F)])

    present = [(cnt_k[c] > 0.0).astype(jnp.float32) for c in range(_L)]
    num_inst = present[0]
    for c in range(1, _L):
        num_inst = num_inst + present[c]
    run = jnp.float32(0.0)
    valid = []
    for c in range(_L):
        run = run + present[c]
        rank = run - 1.0
        valid.append(present[c] * (rank >= 1.0).astype(jnp.float32))

    m = [[jnp.float32(0.0)] * _F]
    for c in range(1, _L):
        denom = jnp.maximum(cnt_k[c], 1.0)
        m.append([sums[c][f] / denom for f in range(_F)])
    msq = []
    for c in range(_L):
        s = m[c][0] * m[c][0]
        for f in range(1, _F):
            s = s + m[c][f] * m[c][f]
        msq.append(s)

    w = [jnp.float32(0.0)]
    for c in range(1, _L):
        w.append(valid[c] / jnp.maximum(cnt_f[c], 1.0))

    msel = [jnp.where(ifl == 1, m[1][f], 0.0) for f in range(_F)]
    msqsel = jnp.where(ifl == 1, msq[1], 0.0)
    wsel = jnp.where(ifl == 1, w[1], 0.0)
    for c in range(2, _L):
        mc = ifl == c
        msel = [jnp.where(mc, m[c][f], msel[f]) for f in range(_F)]
        msqsel = jnp.where(mc, msq[c], msqsel)
        wsel = jnp.where(mc, w[c], wsel)

    dot = e[0] * msel[0]
    esq = e[0] * e[0]
    for f in range(1, _F):
        dot = dot + e[f] * msel[f]
        esq = esq + e[f] * e[f]
    d2 = jnp.maximum(esq - 2.0 * dot + msqsel, 0.0)
    dist = jnp.where(d2 > 0.0, jnp.sqrt(jnp.where(d2 > 0.0, d2, 1.0)), 0.0)
    t = jnp.maximum(dist - _DELTA_AGG, 0.0)
    v = jnp.log(t * t + 1.0)
    agg_num = jnp.sum(v * wsel)
    l_agg = agg_num / jnp.maximum(num_inst - 1.0, 1.0)

    dis_num = jnp.float32(0.0)
    pm_sum = jnp.float32(0.0)
    for i in range(_L):
        for j in range(i + 1, _L):
            dsq = jnp.float32(0.0)
            for f in range(_F):
                dd = m[i][f] - m[j][f]
                dsq = dsq + dd * dd
            d = _scalar_safe_sqrt(dsq)
            vij = jnp.log(jnp.maximum(2.0 * _DELTA_DIS - d, 0.0) ** 2 + 1.0)
            pm_ij = valid[i] * valid[j]
            dis_num = dis_num + 2.0 * vij * pm_ij
            pm_sum = pm_sum + 2.0 * pm_ij
    l_dis = dis_num / jnp.maximum(pm_sum, 1.0)

    reg = jnp.float32(0.0)
    for c in range(_L):
        reg = reg + jnp.log(_scalar_safe_sqrt(msq[c]) + 1.0)
    l_reg = reg / jnp.maximum(num_inst, 1.0) * 0.001

    total = _W_AGG * l_agg + _W_DIS * l_dis + l_reg
    out_ref[0, 0, 0] = jnp.where(num_inst <= 1.0, 0.0, total)


def kernel(emb, instance, kernel, training_mask):
    B, F, H, W = emb.shape
    instance = instance.astype(jnp.int32)
    out = pl.pallas_call(
        _loss_body,
        grid=(B,),
        in_specs=[
            pl.BlockSpec((1, H, W), lambda b: (b, 0, 0)),
            pl.BlockSpec((1, H, W), lambda b: (b, 0, 0)),
            pl.BlockSpec((1, H, W), lambda b: (b, 0, 0)),
            pl.BlockSpec((1, F, H, W), lambda b: (b, 0, 0, 0)),
        ],
        out_specs=pl.BlockSpec(
            (1, 1, 1), lambda b: (b, 0, 0), memory_space=pltpu.SMEM
        ),
        out_shape=jax.ShapeDtypeStruct((B, 1, 1), jnp.float32),
        compiler_params=pltpu.CompilerParams(
            dimension_semantics=("arbitrary",),
        ),
    )(instance, kernel, training_mask, emb)
    return out.reshape(B)


# speedup vs baseline: 7.6271x; 1.3753x over previous
import jax
import jax.numpy as jnp
from jax.experimental import pallas as pl
from jax.experimental.pallas import tpu as pltpu

_F = 4
_L = 8
_DELTA_AGG = 0.5
_DELTA_DIS = 1.5
_W_AGG = 1.0
_W_DIS = 1.0
_RC = 8
_CW = 128


def _scalar_safe_sqrt(sq):
    safe = jnp.where(sq == 0.0, 1.0, sq)
    return jnp.where(sq == 0.0, 0.0, jnp.sqrt(safe))


def _fold(x):
    return (x[:, 0:128] + x[:, 128:256]) + (x[:, 256:384] + x[:, 384:512])


def _loss_body(inst_ref, kern_ref, tm_ref, emb_ref, out_ref):
    nchunks = inst_ref.shape[1]
    zero = jnp.zeros((_RC, _CW), jnp.float32)

    def load_ik(i):
        inst = inst_ref[0, i]
        kn = kern_ref[0, i] > 0.5
        tm = tm_ref[0, i] > 0.5
        return jnp.where(tm & kn, inst, 0)

    acc = {}
    for c in range(1, 5):
        acc[c] = [zero] * (_F + 1)
    for i in range(nchunks):
        ik = load_ik(i)
        e = [emb_ref[0, f, i] for f in range(_F)]
        for c in range(1, 5):
            mk = (ik == c).astype(jnp.float32)
            acc[c][_F] = acc[c][_F] + _fold(mk)
            for f in range(_F):
                acc[c][f] = acc[c][f] + _fold(e[f] * mk)

    for c in range(5, 8):
        acc[c] = [zero] * (_F + 1)
    acc0 = zero
    fcnt = {c: zero for c in range(1, 8)}
    for i in range(nchunks):
        inst = inst_ref[0, i]
        tm = tm_ref[0, i] > 0.5
        kn = kern_ref[0, i] > 0.5
        ik = jnp.where(tm & kn, inst, 0)
        ifl = jnp.where(tm, inst, 0)
        e = [emb_ref[0, f, i] for f in range(_F)]
        for c in range(5, 8):
            mk = (ik == c).astype(jnp.float32)
            acc[c][_F] = acc[c][_F] + _fold(mk)
            for f in range(_F):
                acc[c][f] = acc[c][f] + _fold(e[f] * mk)
        acc0 = acc0 + _fold((ik == 0).astype(jnp.float32))
        for c in range(1, 8):
            fcnt[c] = fcnt[c] + _fold((ifl == c).astype(jnp.float32))

    cnt_k = [jnp.sum(acc0)]
    cnt_f = [jnp.float32(0.0)]
    sums = [[jnp.float32(0.0)] * _F]
    for c in range(1, _L):
        cnt_k.append(jnp.sum(acc[c][_F]))
        cnt_f.append(jnp.sum(fcnt[c]))
        sums.append([jnp.sum(acc[c][f]) for f in range(_F)])

    present = [(cnt_k[c] > 0.0).astype(jnp.float32) for c in range(_L)]
    num_inst = present[0]
    for c in range(1, _L):
        num_inst = num_inst + present[c]
    run = jnp.float32(0.0)
    valid = []
    for c in range(_L):
        run = run + present[c]
        rank = run - 1.0
        valid.append(present[c] * (rank >= 1.0).astype(jnp.float32))

    m = [[jnp.float32(0.0)] * _F]
    for c in range(1, _L):
        denom = jnp.maximum(cnt_k[c], 1.0)
        m.append([sums[c][f] / denom for f in range(_F)])

    w = [jnp.float32(0.0)]
    for c in range(1, _L):
        w.append(valid[c] / jnp.maximum(cnt_f[c], 1.0))

    agg = zero
    for i in range(nchunks):
        inst = inst_ref[0, i]
        tm = tm_ref[0, i] > 0.5
        ifl = jnp.where(tm, inst, 0)
        e = [emb_ref[0, f, i] for f in range(_F)]
        cm = [None] + [ifl == c for c in range(1, _L)]
        msel = [jnp.where(cm[1], m[1][f], 0.0) for f in range(_F)]
        wsel = jnp.where(cm[1], w[1], 0.0)
        for c in range(2, _L):
            msel = [jnp.where(cm[c], m[c][f], msel[f]) for f in range(_F)]
            wsel = jnp.where(cm[c], w[c], wsel)
        dd0 = e[0] - msel[0]
        d2 = dd0 * dd0
        for f in range(1, _F):
            ddf = e[f] - msel[f]
            d2 = d2 + ddf * ddf
        dist = jnp.sqrt(d2)
        t = jnp.maximum(dist - _DELTA_AGG, 0.0)
        v = jnp.log(t * t + 1.0)
        agg = agg + _fold(v * wsel)
    agg_num = jnp.sum(agg)
    l_agg = agg_num / jnp.maximum(num_inst - 1.0, 1.0)

    dis_num = jnp.float32(0.0)
    pm_sum = jnp.float32(0.0)
    for i in range(_L):
        for j in range(i + 1, _L):
            dsq = jnp.float32(0.0)
            for f in range(_F):
                dd = m[i][f] - m[j][f]
                dsq = dsq + dd * dd
            d = _scalar_safe_sqrt(dsq)
            vij = jnp.log(jnp.maximum(2.0 * _DELTA_DIS - d, 0.0) ** 2 + 1.0)
            pm_ij = valid[i] * valid[j]
            dis_num = dis_num + 2.0 * vij * pm_ij
            pm_sum = pm_sum + 2.0 * pm_ij
    l_dis = dis_num / jnp.maximum(pm_sum, 1.0)

    reg = jnp.float32(0.0)
    for c in range(_L):
        msq = jnp.float32(0.0)
        for f in range(_F):
            msq = msq + m[c][f] * m[c][f]
        reg = reg + jnp.log(_scalar_safe_sqrt(msq) + 1.0)
    l_reg = reg / jnp.maximum(num_inst, 1.0) * 0.001

    total = _W_AGG * l_agg + _W_DIS * l_dis + l_reg
    out_ref[0, 0, 0] = jnp.where(num_inst <= 1.0, 0.0, total)


def kernel(emb, instance, kernel, training_mask):
    B, F, H, W = emb.shape
    nch = H // _RC
    instance = instance.astype(jnp.int32).reshape(B, nch, _RC, W)
    kernel = kernel.reshape(B, nch, _RC, W)
    training_mask = training_mask.reshape(B, nch, _RC, W)
    emb = emb.reshape(B, F, nch, _RC, W)
    out = pl.pallas_call(
        _loss_body,
        grid=(B,),
        in_specs=[
            pl.BlockSpec((1, nch, _RC, W), lambda b: (b, 0, 0, 0)),
            pl.BlockSpec((1, nch, _RC, W), lambda b: (b, 0, 0, 0)),
            pl.BlockSpec((1, nch, _RC, W), lambda b: (b, 0, 0, 0)),
            pl.BlockSpec((1, F, nch, _RC, W), lambda b: (b, 0, 0, 0, 0)),
        ],
        out_specs=pl.BlockSpec(
            (1, 1, 1), lambda b: (b, 0, 0), memory_space=pltpu.SMEM
        ),
        out_shape=jax.ShapeDtypeStruct((B, 1, 1), jnp.float32),
        compiler_params=pltpu.CompilerParams(
            dimension_semantics=("arbitrary",),
        ),
    )(instance, kernel, training_mask, emb)
    return out.reshape(B)


# speedup vs baseline: 8.1111x; 1.0635x over previous
import jax
import jax.numpy as jnp
from jax.experimental import pallas as pl
from jax.experimental.pallas import tpu as pltpu

_F = 4
_L = 8
_DELTA_AGG = 0.5
_DELTA_DIS = 1.5
_W_AGG = 1.0
_W_DIS = 1.0
_RC = 8
_CW = 128


def _scalar_safe_sqrt(sq):
    safe = jnp.where(sq == 0.0, 1.0, sq)
    return jnp.where(sq == 0.0, 0.0, jnp.sqrt(safe))


def _fold(x):
    return (x[:, 0:128] + x[:, 128:256]) + (x[:, 256:384] + x[:, 384:512])


def _loss_body(inst_ref, kern_ref, tm_ref, emb_ref, out_ref):
    nchunks = inst_ref.shape[1]
    zero = jnp.zeros((_RC, _CW), jnp.float32)

    def load_ik(i):
        inst = inst_ref[0, i]
        kn = kern_ref[0, i] > 0.5
        tm = tm_ref[0, i] > 0.5
        return jnp.where(tm & kn, inst, 0)

    acc = {}
    for c in range(1, 5):
        acc[c] = [zero] * (_F + 1)
    for i in range(nchunks):
        ik = load_ik(i)
        e = [emb_ref[0, f, i] for f in range(_F)]
        for c in range(1, 5):
            mk = (ik == c).astype(jnp.float32)
            acc[c][_F] = acc[c][_F] + _fold(mk)
            for f in range(_F):
                acc[c][f] = acc[c][f] + _fold(e[f] * mk)

    for c in range(5, 8):
        acc[c] = [zero] * (_F + 1)
    for i in range(nchunks):
        ik = load_ik(i)
        e = [emb_ref[0, f, i] for f in range(_F)]
        for c in range(5, 8):
            mk = (ik == c).astype(jnp.float32)
            acc[c][_F] = acc[c][_F] + _fold(mk)
            for f in range(_F):
                acc[c][f] = acc[c][f] + _fold(e[f] * mk)

    npix = jnp.float32(inst_ref.shape[1] * inst_ref.shape[2] * inst_ref.shape[3])
    cnt_k = [jnp.float32(0.0)]
    sums = [[jnp.float32(0.0)] * _F]
    for c in range(1, _L):
        cnt_k.append(jnp.sum(acc[c][_F]))
        sums.append([jnp.sum(acc[c][f]) for f in range(_F)])
    cnt_k[0] = npix
    for c in range(1, _L):
        cnt_k[0] = cnt_k[0] - cnt_k[c]

    present = [(cnt_k[c] > 0.0).astype(jnp.float32) for c in range(_L)]
    num_inst = present[0]
    for c in range(1, _L):
        num_inst = num_inst + present[c]
    run = jnp.float32(0.0)
    valid = []
    for c in range(_L):
        run = run + present[c]
        rank = run - 1.0
        valid.append(present[c] * (rank >= 1.0).astype(jnp.float32))

    m = [[jnp.float32(0.0)] * _F]
    for c in range(1, _L):
        denom = jnp.maximum(cnt_k[c], 1.0)
        m.append([sums[c][f] / denom for f in range(_F)])

    agg = {c: zero for c in range(1, _L)}
    fcnt = {c: zero for c in range(1, _L)}
    for i in range(nchunks):
        inst = inst_ref[0, i]
        tm = tm_ref[0, i] > 0.5
        ifl = jnp.where(tm, inst, 0)
        e = [emb_ref[0, f, i] for f in range(_F)]
        cm = [None] + [ifl == c for c in range(1, _L)]
        msel = [jnp.where(cm[1], m[1][f], 0.0) for f in range(_F)]
        for c in range(2, _L):
            msel = [jnp.where(cm[c], m[c][f], msel[f]) for f in range(_F)]
        dd0 = e[0] - msel[0]
        d2 = dd0 * dd0
        for f in range(1, _F):
            ddf = e[f] - msel[f]
            d2 = d2 + ddf * ddf
        dist = jnp.sqrt(d2)
        t = jnp.maximum(dist - _DELTA_AGG, 0.0)
        v = jnp.log(t * t + 1.0)
        for c in range(1, _L):
            agg[c] = agg[c] + _fold(jnp.where(cm[c], v, 0.0))
            fcnt[c] = fcnt[c] + _fold(cm[c].astype(jnp.float32))
    l_agg = jnp.float32(0.0)
    for c in range(1, _L):
        l_agg = l_agg + valid[c] * jnp.sum(agg[c]) / jnp.maximum(
            jnp.sum(fcnt[c]), 1.0)
    l_agg = l_agg / jnp.maximum(num_inst - 1.0, 1.0)

    dis_num = jnp.float32(0.0)
    pm_sum = jnp.float32(0.0)
    for i in range(_L):
        for j in range(i + 1, _L):
            dsq = jnp.float32(0.0)
            for f in range(_F):
                dd = m[i][f] - m[j][f]
                dsq = dsq + dd * dd
            d = _scalar_safe_sqrt(dsq)
            vij = jnp.log(jnp.maximum(2.0 * _DELTA_DIS - d, 0.0) ** 2 + 1.0)
            pm_ij = valid[i] * valid[j]
            dis_num = dis_num + 2.0 * vij * pm_ij
            pm_sum = pm_sum + 2.0 * pm_ij
    l_dis = dis_num / jnp.maximum(pm_sum, 1.0)

    reg = jnp.float32(0.0)
    for c in range(_L):
        msq = jnp.float32(0.0)
        for f in range(_F):
            msq = msq + m[c][f] * m[c][f]
        reg = reg + jnp.log(_scalar_safe_sqrt(msq) + 1.0)
    l_reg = reg / jnp.maximum(num_inst, 1.0) * 0.001

    total = _W_AGG * l_agg + _W_DIS * l_dis + l_reg
    out_ref[0, 0, 0] = jnp.where(num_inst <= 1.0, 0.0, total)


def kernel(emb, instance, kernel, training_mask):
    B, F, H, W = emb.shape
    nch = H // _RC
    instance = instance.astype(jnp.int32).reshape(B, nch, _RC, W)
    kernel = kernel.reshape(B, nch, _RC, W)
    training_mask = training_mask.reshape(B, nch, _RC, W)
    emb = emb.reshape(B, F, nch, _RC, W)
    out = pl.pallas_call(
        _loss_body,
        grid=(B,),
        in_specs=[
            pl.BlockSpec((1, nch, _RC, W), lambda b: (b, 0, 0, 0)),
            pl.BlockSpec((1, nch, _RC, W), lambda b: (b, 0, 0, 0)),
            pl.BlockSpec((1, nch, _RC, W), lambda b: (b, 0, 0, 0)),
            pl.BlockSpec((1, F, nch, _RC, W), lambda b: (b, 0, 0, 0, 0)),
        ],
        out_specs=pl.BlockSpec(
            (1, 1, 1), lambda b: (b, 0, 0), memory_space=pltpu.SMEM
        ),
        out_shape=jax.ShapeDtypeStruct((B, 1, 1), jnp.float32),
        compiler_params=pltpu.CompilerParams(
            dimension_semantics=("arbitrary",),
        ),
    )(instance, kernel, training_mask, emb)
    return out.reshape(B)


# speedup vs baseline: 8.4767x; 1.0451x over previous
import jax
import jax.numpy as jnp
from jax.experimental import pallas as pl
from jax.experimental.pallas import tpu as pltpu

_F = 4
_L = 8
_DELTA_AGG = 0.5
_DELTA_DIS = 1.5
_W_AGG = 1.0
_W_DIS = 1.0
_RC = 8
_CW = 128


def _scalar_safe_sqrt(sq):
    safe = jnp.where(sq == 0.0, 1.0, sq)
    return jnp.where(sq == 0.0, 0.0, jnp.sqrt(safe))


def _fold(x):
    return (x[:, 0:128] + x[:, 128:256]) + (x[:, 256:384] + x[:, 384:512])


def _loss_body(inst_ref, kern_ref, tm_ref, emb_ref, out_ref):
    nchunks = inst_ref.shape[1]
    zero = jnp.zeros((_RC, _CW), jnp.float32)

    def load_ik(i):
        inst = inst_ref[0, i]
        kn = kern_ref[0, i] > 0.5
        tm = tm_ref[0, i] > 0.5
        return jnp.where(tm & kn, inst, 0)

    acc = {}
    for c in range(1, 5):
        acc[c] = [zero] * (_F + 1)
    for i in range(nchunks):
        ik = load_ik(i)
        e = [emb_ref[0, f, i] for f in range(_F)]
        for c in range(1, 5):
            mk = (ik == c).astype(jnp.float32)
            acc[c][_F] = acc[c][_F] + _fold(mk)
            for f in range(_F):
                acc[c][f] = acc[c][f] + _fold(e[f] * mk)

    for c in range(5, 8):
        acc[c] = [zero] * (_F + 1)
    for i in range(nchunks):
        ik = load_ik(i)
        e = [emb_ref[0, f, i] for f in range(_F)]
        for c in range(5, 8):
            mk = (ik == c).astype(jnp.float32)
            acc[c][_F] = acc[c][_F] + _fold(mk)
            for f in range(_F):
                acc[c][f] = acc[c][f] + _fold(e[f] * mk)

    npix = jnp.float32(inst_ref.shape[1] * inst_ref.shape[2] * inst_ref.shape[3])
    cnt_k = [jnp.float32(0.0)]
    sums = [[jnp.float32(0.0)] * _F]
    for c in range(1, _L):
        cnt_k.append(jnp.sum(acc[c][_F]))
        sums.append([jnp.sum(acc[c][f]) for f in range(_F)])
    cnt_k[0] = npix
    for c in range(1, _L):
        cnt_k[0] = cnt_k[0] - cnt_k[c]

    present = [(cnt_k[c] > 0.0).astype(jnp.float32) for c in range(_L)]
    num_inst = present[0]
    for c in range(1, _L):
        num_inst = num_inst + present[c]
    run = jnp.float32(0.0)
    valid = []
    for c in range(_L):
        run = run + present[c]
        rank = run - 1.0
        valid.append(present[c] * (rank >= 1.0).astype(jnp.float32))

    m = [[jnp.float32(0.0)] * _F]
    for c in range(1, _L):
        denom = jnp.maximum(cnt_k[c], 1.0)
        m.append([sums[c][f] / denom for f in range(_F)])

    agg = {c: zero for c in range(1, _L)}
    fcnt = {c: zero for c in range(1, _L)}
    for i in range(nchunks):
        inst = inst_ref[0, i]
        tm = tm_ref[0, i] > 0.5
        ifl = jnp.where(tm, inst, 0)
        e = [emb_ref[0, f, i] for f in range(_F)]
        cm = [None] + [ifl == c for c in range(1, _L)]
        msel = [jnp.where(cm[1], m[1][f], 0.0) for f in range(_F)]
        for c in range(2, _L):
            msel = [jnp.where(cm[c], m[c][f], msel[f]) for f in range(_F)]
        dd0 = e[0] - msel[0]
        d2 = dd0 * dd0
        for f in range(1, _F):
            ddf = e[f] - msel[f]
            d2 = d2 + ddf * ddf
        dist = jnp.sqrt(d2)
        t = jnp.maximum(dist - _DELTA_AGG, 0.0)
        v = jnp.log(t * t + 1.0)
        for c in range(1, _L):
            agg[c] = agg[c] + _fold(jnp.where(cm[c], v, 0.0))
            fcnt[c] = fcnt[c] + _fold(cm[c].astype(jnp.float32))
    l_agg = jnp.float32(0.0)
    for c in range(1, _L):
        l_agg = l_agg + valid[c] * jnp.sum(agg[c]) / jnp.maximum(
            jnp.sum(fcnt[c]), 1.0)
    l_agg = l_agg / jnp.maximum(num_inst - 1.0, 1.0)

    dsq_list = []
    pm_list = []
    for i in range(_L):
        for j in range(i + 1, _L):
            dsq = jnp.float32(0.0)
            for f in range(_F):
                dd = m[i][f] - m[j][f]
                dsq = dsq + dd * dd
            dsq_list.append(dsq)
            pm_list.append(valid[i] * valid[j])
    msq_list = []
    for c in range(_L):
        msq = jnp.float32(0.0)
        for f in range(_F):
            msq = msq + m[c][f] * m[c][f]
        msq_list.append(msq)

    npair = len(dsq_list)
    sq = jnp.stack(dsq_list + msq_list)
    d = jnp.where(sq == 0.0, 0.0,
                  jnp.sqrt(jnp.where(sq == 0.0, 1.0, sq)))
    lane = jax.lax.iota(jnp.int32, npair + _L)
    is_pair = lane < npair
    tdis = jnp.maximum(2.0 * _DELTA_DIS - d, 0.0)
    arg = jnp.where(is_pair, tdis * tdis + 1.0, d + 1.0)
    v = jnp.log(arg)
    pmv = jnp.stack(pm_list + [jnp.float32(0.0)] * _L)
    dis_num = 2.0 * jnp.sum(v * pmv)
    pm_sum = 2.0 * jnp.sum(pmv)
    reg_sum = jnp.sum(jnp.where(is_pair, 0.0, v))
    l_dis = dis_num / jnp.maximum(pm_sum, 1.0)
    l_reg = reg_sum / jnp.maximum(num_inst, 1.0) * 0.001

    total = _W_AGG * l_agg + _W_DIS * l_dis + l_reg
    out_ref[0, 0, 0] = jnp.where(num_inst <= 1.0, 0.0, total)


def kernel(emb, instance, kernel, training_mask):
    B, F, H, W = emb.shape
    nch = H // _RC
    instance = instance.astype(jnp.int32).reshape(B, nch, _RC, W)
    kernel = kernel.reshape(B, nch, _RC, W)
    training_mask = training_mask.reshape(B, nch, _RC, W)
    emb = emb.reshape(B, F, nch, _RC, W)
    out = pl.pallas_call(
        _loss_body,
        grid=(B,),
        in_specs=[
            pl.BlockSpec((1, nch, _RC, W), lambda b: (b, 0, 0, 0)),
            pl.BlockSpec((1, nch, _RC, W), lambda b: (b, 0, 0, 0)),
            pl.BlockSpec((1, nch, _RC, W), lambda b: (b, 0, 0, 0)),
            pl.BlockSpec((1, F, nch, _RC, W), lambda b: (b, 0, 0, 0, 0)),
        ],
        out_specs=pl.BlockSpec(
            (1, 1, 1), lambda b: (b, 0, 0), memory_space=pltpu.SMEM
        ),
        out_shape=jax.ShapeDtypeStruct((B, 1, 1), jnp.float32),
        compiler_params=pltpu.CompilerParams(
            dimension_semantics=("arbitrary",),
        ),
    )(instance, kernel, training_mask, emb)
    return out.reshape(B)
